# baseline (speedup 1.0000x reference)
import functools

import jax
import jax.numpy as jnp
from jax import lax
from jax.experimental import pallas as pl
from jax.experimental.pallas import tpu as pltpu
from jax.experimental.pallas import tpu_sc as plsc

N_NODES = 10000
D = 128

NC = 2
NS = 16
NW = NC * NS
N_PAD = 10240
ROWS_PER_SUB = N_PAD // NS
KC = 128
E_PAD = 327680
DUMMY = N_NODES

_GB = 8
_NBLK0 = 10
_NBLK1 = 0
_NCH0 = _NBLK0 * _GB
_NCH1 = _NBLK1 * _GB
CHUNKS = E_PAD // KC
_C1_BASE = NS * _NCH0

EW = E_PAD // NW
NCHUNK = EW // KC

_MESH = plsc.VectorSubcoreMesh(core_axis_name="c", subcore_axis_name="s")


@functools.partial(
    pl.kernel,
    out_type=jax.ShapeDtypeStruct((NC * N_PAD, D), jnp.float32),
    mesh=_MESH,
    scratch_types=[
        pltpu.VMEM((NCHUNK, KC), jnp.int32),
        pltpu.VMEM((KC, D), jnp.float32),
        pltpu.VMEM_SHARED((N_PAD, D), jnp.float32),
    ],
)
def _deg_kernel(dst_hbm, out_hbm, dstv, valv, deg_sh):
    c = lax.axis_index("c")
    s = lax.axis_index("s")
    wid = c * NS + s

    pltpu.sync_copy(dst_hbm.at[pl.ds(wid * NCHUNK, NCHUNK)], dstv)

    @pl.loop(0, KC)
    def _zero_val(i):
        @pl.loop(0, D // 16)
        def _zv(j):
            valv[i, pl.ds(j * 16, 16)] = jnp.zeros((16,), jnp.float32)

    @pl.loop(0, ROWS_PER_SUB // KC)
    def _zero_acc(k):
        pltpu.sync_copy(valv, deg_sh.at[pl.ds(s * ROWS_PER_SUB + k * KC, KC)])

    @pl.loop(0, KC)
    def _ones_val(i):
        @pl.loop(0, D // 16)
        def _ov(j):
            valv[i, pl.ds(j * 16, 16)] = jnp.ones((16,), jnp.float32)

    plsc.subcore_barrier()

    @pl.loop(0, NCHUNK)
    def _accumulate(j):
        pltpu.sync_copy(valv, deg_sh.at[dstv.at[j]], add=True)

    plsc.subcore_barrier()
    pltpu.sync_copy(
        deg_sh.at[pl.ds(s * ROWS_PER_SUB, ROWS_PER_SUB)],
        out_hbm.at[pl.ds(c * N_PAD + s * ROWS_PER_SUB, ROWS_PER_SUB)],
    )


@functools.partial(
    pl.kernel,
    out_type=jax.ShapeDtypeStruct((NC * N_PAD, D), jnp.float32),
    mesh=_MESH,
    scratch_types=(
        [pltpu.VMEM((_GB, 2, KC), jnp.int32) for _ in range(2)]
        + [pltpu.VMEM((KC, D), jnp.float32) for _ in range(2)]
        + [pltpu.SemaphoreType.DMA for _ in range(4)]
        + [pltpu.VMEM_SHARED((N_PAD, D), jnp.float32)]
    ),
)
def _agg_kernel(table_hbm, idx_hbm, out_hbm, ib0, ib1, r0, r1,
                isem0, isem1, gsem0, gsem1, acc_sh):
    idxb = (ib0, ib1)
    isem = (isem0, isem1)
    rows = (r0, r1)
    gsem = (gsem0, gsem1)

    c = lax.axis_index("c")
    s = lax.axis_index("s")

    @pl.loop(0, KC)
    def _zero_rows(i):
        @pl.loop(0, D // 16)
        def _zero_lane(j):
            rows[0][i, pl.ds(j * 16, 16)] = jnp.zeros((16,), jnp.float32)

    @pl.loop(0, ROWS_PER_SUB // KC)
    def _zero_acc(k):
        pltpu.sync_copy(rows[0], acc_sh.at[pl.ds(s * ROWS_PER_SUB + k * KC, KC)])

    plsc.subcore_barrier()

    def _run(cb, nblk):
        def iblk(k):
            return idx_hbm.at[pl.ds(cb + k * _GB, _GB)]

        for p in range(2):
            pltpu.async_copy(iblk(p), idxb[p], isem[p])
        pltpu.make_async_copy(iblk(0), idxb[0], isem[0]).wait()
        for b in range(2):
            pltpu.async_copy(table_hbm.at[idxb[0].at[b, 0]], rows[b], gsem[b])

        def _block(k, p, has_next, prefetch, tail):
            q = 1 - p
            if has_next:
                pltpu.make_async_copy(iblk(k + 1), idxb[q], isem[q]).wait()
            for b in range(_GB):
                rb = rows[b % 2]
                pltpu.make_async_copy(
                    table_hbm.at[idxb[p].at[b, 0]], rb, gsem[b % 2]).wait()
                pltpu.sync_copy(rb, acc_sh.at[idxb[p].at[b, 1]], add=True)
                if not (tail and b >= _GB - 2):
                    if b < _GB - 2:
                        nsrc = idxb[p].at[b + 2, 0]
                    else:
                        nsrc = idxb[q].at[b - (_GB - 2), 0]
                    pltpu.async_copy(table_hbm.at[nsrc], rb, gsem[b % 2])
            if prefetch:
                pltpu.async_copy(iblk(k + 2), idxb[p], isem[p])

        @pl.loop(0, nblk - 2, step=2)
        def _blocks(m):
            _block(m, 0, True, True, False)
            _block(m + 1, 1, True, True, False)

        _block(nblk - 2, 0, True, False, False)
        _block(nblk - 1, 1, False, False, True)

    @pl.when(c == 0)
    def _core0():
        _run(s * _NCH0, _NBLK0)

    plsc.subcore_barrier()
    pltpu.sync_copy(
        acc_sh.at[pl.ds(s * ROWS_PER_SUB, ROWS_PER_SUB)],
        out_hbm.at[pl.ds(c * N_PAD + s * ROWS_PER_SUB, ROWS_PER_SUB)],
    )


_RB = 1000


def _dinv_from_parts(degp):
    deg = degp[0, :, 0:1] + degp[1, :, 0:1]
    return jnp.where(deg > 0, lax.rsqrt(jnp.maximum(deg, 1e-12)), 0.0)


def _mm1_body(x_ref, w_ref, degp_ref, o_ref):
    dinv = _dinv_from_parts(degp_ref[...])
    h = jnp.dot(x_ref[...], w_ref[...], preferred_element_type=jnp.float32)
    o_ref[...] = h * dinv


def _mm1(x, W1, degp):
    return pl.pallas_call(
        _mm1_body,
        grid=(N_NODES // _RB,),
        in_specs=[
            pl.BlockSpec((_RB, D), lambda i: (i, 0)),
            pl.BlockSpec((D, D), lambda i: (0, 0)),
            pl.BlockSpec((NC, _RB, D), lambda i: (0, i, 0)),
        ],
        out_specs=pl.BlockSpec((_RB, D), lambda i: (i, 0)),
        out_shape=jax.ShapeDtypeStruct((N_NODES, D), jnp.float32),
    )(x, W1, degp)


def _mid_body(p_ref, p2_ref, degp_ref, b_ref, w_ref, o_ref):
    dinv = _dinv_from_parts(degp_ref[...])
    agg = p_ref[0] + p_ref[1] + p2_ref[0] + p2_ref[1]
    h = jax.nn.relu(agg * dinv + b_ref[...])
    o_ref[...] = (
        jnp.dot(h, w_ref[...], preferred_element_type=jnp.float32) * dinv
    )


def _mid(P, P2, degp, b1, W2):
    return pl.pallas_call(
        _mid_body,
        grid=(N_NODES // _RB,),
        in_specs=[
            pl.BlockSpec((NC, _RB, D), lambda i: (0, i, 0)),
            pl.BlockSpec((NC, _RB, D), lambda i: (0, i, 0)),
            pl.BlockSpec((NC, _RB, D), lambda i: (0, i, 0)),
            pl.BlockSpec((1, D), lambda i: (0, 0)),
            pl.BlockSpec((D, D), lambda i: (0, 0)),
        ],
        out_specs=pl.BlockSpec((_RB, D), lambda i: (i, 0)),
        out_shape=jax.ShapeDtypeStruct((N_NODES, D), jnp.float32),
    )(P, P2, degp, b1, W2)


def _fin_body(q_ref, q2_ref, degp_ref, b_ref, wih_ref, bih_ref, bhh_ref,
              o_ref):
    dinv = _dinv_from_parts(degp_ref[...])
    agg = q_ref[0] + q_ref[1] + q2_ref[0] + q2_ref[1]
    h = jax.nn.relu(agg * dinv + b_ref[...])
    g = jnp.dot(h, wih_ref[...], preferred_element_type=jnp.float32)
    g = g + bih_ref[...]
    xr = g[:, 0:D]
    xz = g[:, D:2 * D]
    xn = g[:, 2 * D:3 * D]
    hr = bhh_ref[:, 0:D]
    hz = bhh_ref[:, D:2 * D]
    hn = bhh_ref[:, 2 * D:3 * D]
    r = jax.nn.sigmoid(xr + hr)
    z = jax.nn.sigmoid(xz + hz)
    n = jnp.tanh(xn + r * hn)
    o_ref[...] = (1.0 - z) * n


def _fin(Q, Q2, degp, b2, W_ih, b_ih, b_hh):
    return pl.pallas_call(
        _fin_body,
        grid=(N_NODES // _RB,),
        in_specs=[
            pl.BlockSpec((NC, _RB, D), lambda i: (0, i, 0)),
            pl.BlockSpec((NC, _RB, D), lambda i: (0, i, 0)),
            pl.BlockSpec((NC, _RB, D), lambda i: (0, i, 0)),
            pl.BlockSpec((1, D), lambda i: (0, 0)),
            pl.BlockSpec((D, 3 * D), lambda i: (0, 0)),
            pl.BlockSpec((1, 3 * D), lambda i: (0, 0)),
            pl.BlockSpec((1, 3 * D), lambda i: (0, 0)),
        ],
        out_specs=pl.BlockSpec((_RB, D), lambda i: (i, 0)),
        out_shape=jax.ShapeDtypeStruct((N_NODES, D), jnp.float32),
    )(Q, Q2, degp, b2, W_ih, b_ih, b_hh)


def kernel(x, A, W1, b1, W2, b2, W_ih, W_hh, b_ih, b_hh):
    A = A.astype(jnp.int32)
    src = A[0]
    dst = A[1]
    pad = E_PAD - src.shape[0]
    pad_dst = DUMMY + jnp.arange(pad, dtype=jnp.int32) % (N_PAD - N_NODES)
    src_p = jnp.concatenate([src, jnp.zeros((pad,), jnp.int32)])
    dst_p = jnp.concatenate([dst, pad_dst])
    idx = jnp.stack([src_p.reshape(CHUNKS, KC),
                     dst_p.reshape(CHUNKS, KC)], axis=1)
    dst_p = dst_p.reshape(CHUNKS, KC)

    degp = _deg_kernel(dst_p).reshape(NC, N_PAD, D)

    idxA = idx[:CHUNKS // 2]
    idxB = idx[CHUNKS // 2:]
    hs1 = _mm1(x, W1, degp)
    PA = _agg_kernel(hs1, idxA).reshape(NC, N_PAD, D)
    PB = _agg_kernel(hs1, idxB).reshape(NC, N_PAD, D)
    hs2 = _mid(PA, PB, degp, b1.reshape(1, D), W2)
    QA = _agg_kernel(hs2, idxA).reshape(NC, N_PAD, D)
    QB = _agg_kernel(hs2, idxB).reshape(NC, N_PAD, D)
    out = _fin(QA, QB, degp, b2.reshape(1, D), W_ih, b_ih.reshape(1, 3 * D),
               b_hh.reshape(1, 3 * D))
    return out


# speedup vs baseline: 1.2048x; 1.2048x over previous
import functools

import jax
import jax.numpy as jnp
from jax import lax
from jax.experimental import pallas as pl
from jax.experimental.pallas import tpu as pltpu
from jax.experimental.pallas import tpu_sc as plsc

N_NODES = 10000
D = 128

NC = 2
NS = 16
NW = NC * NS
N_PAD = 10240
ROWS_PER_SUB = N_PAD // NS
KC = 128
E_PAD = 327680
DUMMY = N_NODES

_GB = 8
_NBLK0 = 10
_NBLK1 = 10
_NCH0 = _NBLK0 * _GB
_NCH1 = _NBLK1 * _GB
CHUNKS = E_PAD // KC
_C1_BASE = NS * _NCH0

EW = E_PAD // NW
NCHUNK = EW // KC

_MESH = plsc.VectorSubcoreMesh(core_axis_name="c", subcore_axis_name="s")


@functools.partial(
    pl.kernel,
    out_type=jax.ShapeDtypeStruct((NC * N_PAD, D), jnp.float32),
    mesh=_MESH,
    scratch_types=[
        pltpu.VMEM((NCHUNK, KC), jnp.int32),
        pltpu.VMEM((KC, D), jnp.float32),
        pltpu.VMEM_SHARED((N_PAD, D), jnp.float32),
    ],
)
def _deg_kernel(dst_hbm, out_hbm, dstv, valv, deg_sh):
    c = lax.axis_index("c")
    s = lax.axis_index("s")
    wid = c * NS + s

    pltpu.sync_copy(dst_hbm.at[pl.ds(wid * NCHUNK, NCHUNK)], dstv)

    @pl.loop(0, KC)
    def _zero_val(i):
        @pl.loop(0, D // 16)
        def _zv(j):
            valv[i, pl.ds(j * 16, 16)] = jnp.zeros((16,), jnp.float32)

    @pl.loop(0, ROWS_PER_SUB // KC)
    def _zero_acc(k):
        pltpu.sync_copy(valv, deg_sh.at[pl.ds(s * ROWS_PER_SUB + k * KC, KC)])

    @pl.loop(0, KC)
    def _ones_val(i):
        @pl.loop(0, D // 16)
        def _ov(j):
            valv[i, pl.ds(j * 16, 16)] = jnp.ones((16,), jnp.float32)

    plsc.subcore_barrier()

    @pl.loop(0, NCHUNK)
    def _accumulate(j):
        pltpu.sync_copy(valv, deg_sh.at[dstv.at[j]], add=True)

    plsc.subcore_barrier()
    pltpu.sync_copy(
        deg_sh.at[pl.ds(s * ROWS_PER_SUB, ROWS_PER_SUB)],
        out_hbm.at[pl.ds(c * N_PAD + s * ROWS_PER_SUB, ROWS_PER_SUB)],
    )


@functools.partial(
    pl.kernel,
    out_type=jax.ShapeDtypeStruct((NC * N_PAD, D), jnp.float32),
    mesh=_MESH,
    scratch_types=(
        [pltpu.VMEM((_GB, 2, KC), jnp.int32) for _ in range(2)]
        + [pltpu.VMEM((KC, D), jnp.float32) for _ in range(2)]
        + [pltpu.SemaphoreType.DMA for _ in range(4)]
        + [pltpu.VMEM_SHARED((N_PAD, D), jnp.float32)]
    ),
)
def _agg_kernel(table_hbm, idx_hbm, out_hbm, ib0, ib1, r0, r1,
                isem0, isem1, gsem0, gsem1, acc_sh):
    idxb = (ib0, ib1)
    isem = (isem0, isem1)
    rows = (r0, r1)
    gsem = (gsem0, gsem1)

    c = lax.axis_index("c")
    s = lax.axis_index("s")

    @pl.loop(0, KC)
    def _zero_rows(i):
        @pl.loop(0, D // 16)
        def _zero_lane(j):
            rows[0][i, pl.ds(j * 16, 16)] = jnp.zeros((16,), jnp.float32)

    @pl.loop(0, ROWS_PER_SUB // KC)
    def _zero_acc(k):
        pltpu.sync_copy(rows[0], acc_sh.at[pl.ds(s * ROWS_PER_SUB + k * KC, KC)])

    plsc.subcore_barrier()

    def _run(cb, nblk):
        def iblk(k):
            return idx_hbm.at[pl.ds(cb + k * _GB, _GB)]

        for p in range(2):
            pltpu.async_copy(iblk(p), idxb[p], isem[p])
        pltpu.make_async_copy(iblk(0), idxb[0], isem[0]).wait()
        for b in range(2):
            pltpu.async_copy(table_hbm.at[idxb[0].at[b, 0]], rows[b], gsem[b])

        def _block(k, p, has_next, prefetch, tail):
            q = 1 - p
            if has_next:
                pltpu.make_async_copy(iblk(k + 1), idxb[q], isem[q]).wait()
            for b in range(_GB):
                rb = rows[b % 2]
                pltpu.make_async_copy(
                    table_hbm.at[idxb[p].at[b, 0]], rb, gsem[b % 2]).wait()
                pltpu.sync_copy(rb, acc_sh.at[idxb[p].at[b, 1]], add=True)
                if not (tail and b >= _GB - 2):
                    if b < _GB - 2:
                        nsrc = idxb[p].at[b + 2, 0]
                    else:
                        nsrc = idxb[q].at[b - (_GB - 2), 0]
                    pltpu.async_copy(table_hbm.at[nsrc], rb, gsem[b % 2])
            if prefetch:
                pltpu.async_copy(iblk(k + 2), idxb[p], isem[p])

        @pl.loop(0, nblk - 2, step=2)
        def _blocks(m):
            _block(m, 0, True, True, False)
            _block(m + 1, 1, True, True, False)

        _block(nblk - 2, 0, True, False, False)
        _block(nblk - 1, 1, False, False, True)

    @pl.when(c == 0)
    def _core0():
        _run(s * _NCH0, _NBLK0)

    @pl.when(c == 1)
    def _core1():
        _run(_C1_BASE + s * _NCH1, _NBLK1)

    plsc.subcore_barrier()
    pltpu.sync_copy(
        acc_sh.at[pl.ds(s * ROWS_PER_SUB, ROWS_PER_SUB)],
        out_hbm.at[pl.ds(c * N_PAD + s * ROWS_PER_SUB, ROWS_PER_SUB)],
    )


_RB = 1000


def _dinv_from_parts(degp):
    deg = degp[0, :, 0:1] + degp[1, :, 0:1]
    return jnp.where(deg > 0, lax.rsqrt(jnp.maximum(deg, 1e-12)), 0.0)


def _mm1_body(x_ref, w_ref, degp_ref, o_ref):
    dinv = _dinv_from_parts(degp_ref[...])
    h = jnp.dot(x_ref[...], w_ref[...], preferred_element_type=jnp.float32)
    o_ref[...] = h * dinv


def _mm1(x, W1, degp):
    return pl.pallas_call(
        _mm1_body,
        grid=(N_NODES // _RB,),
        in_specs=[
            pl.BlockSpec((_RB, D), lambda i: (i, 0)),
            pl.BlockSpec((D, D), lambda i: (0, 0)),
            pl.BlockSpec((NC, _RB, D), lambda i: (0, i, 0)),
        ],
        out_specs=pl.BlockSpec((_RB, D), lambda i: (i, 0)),
        out_shape=jax.ShapeDtypeStruct((N_NODES, D), jnp.float32),
    )(x, W1, degp)


def _mid_body(p_ref, p2_ref, degp_ref, b_ref, w_ref, o_ref):
    dinv = _dinv_from_parts(degp_ref[...])
    agg = p_ref[0] + p_ref[1]
    h = jax.nn.relu(agg * dinv + b_ref[...])
    o_ref[...] = (
        jnp.dot(h, w_ref[...], preferred_element_type=jnp.float32) * dinv
    )


def _mid(P, P2, degp, b1, W2):
    return pl.pallas_call(
        _mid_body,
        grid=(N_NODES // _RB,),
        in_specs=[
            pl.BlockSpec((NC, _RB, D), lambda i: (0, i, 0)),
            pl.BlockSpec((NC, _RB, D), lambda i: (0, i, 0)),
            pl.BlockSpec((NC, _RB, D), lambda i: (0, i, 0)),
            pl.BlockSpec((1, D), lambda i: (0, 0)),
            pl.BlockSpec((D, D), lambda i: (0, 0)),
        ],
        out_specs=pl.BlockSpec((_RB, D), lambda i: (i, 0)),
        out_shape=jax.ShapeDtypeStruct((N_NODES, D), jnp.float32),
    )(P, P2, degp, b1, W2)


def _fin_body(q_ref, q2_ref, degp_ref, b_ref, wih_ref, bih_ref, bhh_ref,
              o_ref):
    dinv = _dinv_from_parts(degp_ref[...])
    agg = q_ref[0] + q_ref[1]
    h = jax.nn.relu(agg * dinv + b_ref[...])
    g = jnp.dot(h, wih_ref[...], preferred_element_type=jnp.float32)
    g = g + bih_ref[...]
    xr = g[:, 0:D]
    xz = g[:, D:2 * D]
    xn = g[:, 2 * D:3 * D]
    hr = bhh_ref[:, 0:D]
    hz = bhh_ref[:, D:2 * D]
    hn = bhh_ref[:, 2 * D:3 * D]
    r = jax.nn.sigmoid(xr + hr)
    z = jax.nn.sigmoid(xz + hz)
    n = jnp.tanh(xn + r * hn)
    o_ref[...] = (1.0 - z) * n


def _fin(Q, Q2, degp, b2, W_ih, b_ih, b_hh):
    return pl.pallas_call(
        _fin_body,
        grid=(N_NODES // _RB,),
        in_specs=[
            pl.BlockSpec((NC, _RB, D), lambda i: (0, i, 0)),
            pl.BlockSpec((NC, _RB, D), lambda i: (0, i, 0)),
            pl.BlockSpec((NC, _RB, D), lambda i: (0, i, 0)),
            pl.BlockSpec((1, D), lambda i: (0, 0)),
            pl.BlockSpec((D, 3 * D), lambda i: (0, 0)),
            pl.BlockSpec((1, 3 * D), lambda i: (0, 0)),
            pl.BlockSpec((1, 3 * D), lambda i: (0, 0)),
        ],
        out_specs=pl.BlockSpec((_RB, D), lambda i: (i, 0)),
        out_shape=jax.ShapeDtypeStruct((N_NODES, D), jnp.float32),
    )(Q, Q2, degp, b2, W_ih, b_ih, b_hh)


def kernel(x, A, W1, b1, W2, b2, W_ih, W_hh, b_ih, b_hh):
    A = A.astype(jnp.int32)
    src = A[0]
    dst = A[1]
    pad = E_PAD - src.shape[0]
    pad_dst = DUMMY + jnp.arange(pad, dtype=jnp.int32) % (N_PAD - N_NODES)
    src_p = jnp.concatenate([src, jnp.zeros((pad,), jnp.int32)])
    dst_p = jnp.concatenate([dst, pad_dst])
    idx = jnp.stack([src_p.reshape(CHUNKS, KC),
                     dst_p.reshape(CHUNKS, KC)], axis=1)
    dst_p = dst_p.reshape(CHUNKS, KC)

    degp = _deg_kernel(dst_p).reshape(NC, N_PAD, D)

    hs1 = _mm1(x, W1, degp)
    P = _agg_kernel(hs1, idx).reshape(NC, N_PAD, D)
    hs2 = _mid(P, P, degp, b1.reshape(1, D), W2)
    Q = _agg_kernel(hs2, idx).reshape(NC, N_PAD, D)
    out = _fin(Q, Q, degp, b2.reshape(1, D), W_ih, b_ih.reshape(1, 3 * D),
               b_hh.reshape(1, 3 * D))
    return out


# speedup vs baseline: 3.5437x; 2.9413x over previous
import functools

import jax
import jax.numpy as jnp
from jax import lax
from jax.experimental import pallas as pl
from jax.experimental.pallas import tpu as pltpu
from jax.experimental.pallas import tpu_sc as plsc

N_NODES = 10000
D = 128

NC = 2
NS = 16
NW = NC * NS
N_PAD = 10240
ROWS_PER_SUB = N_PAD // NS
KC = 128
E_PAD = 327680
DUMMY = N_NODES

_GB = 8
_NBLK0 = 10
_NBLK1 = 10
_NCH0 = _NBLK0 * _GB
_NCH1 = _NBLK1 * _GB
CHUNKS = E_PAD // KC
_C1_BASE = NS * _NCH0

EW = E_PAD // NW
NCHUNK = EW // KC

_MESH = plsc.VectorSubcoreMesh(core_axis_name="c", subcore_axis_name="s")


@functools.partial(
    pl.kernel,
    out_type=jax.ShapeDtypeStruct((NC * N_PAD, D), jnp.float32),
    mesh=_MESH,
    scratch_types=[
        pltpu.VMEM((NCHUNK, KC), jnp.int32),
        pltpu.VMEM((KC, D), jnp.float32),
        pltpu.VMEM_SHARED((N_PAD, D), jnp.float32),
    ],
)
def _deg_kernel(dst_hbm, out_hbm, dstv, valv, deg_sh):
    c = lax.axis_index("c")
    s = lax.axis_index("s")
    wid = c * NS + s

    pltpu.sync_copy(dst_hbm.at[pl.ds(wid * NCHUNK, NCHUNK)], dstv)

    @pl.loop(0, KC)
    def _zero_val(i):
        @pl.loop(0, D // 16)
        def _zv(j):
            valv[i, pl.ds(j * 16, 16)] = jnp.zeros((16,), jnp.float32)

    @pl.loop(0, ROWS_PER_SUB // KC)
    def _zero_acc(k):
        pltpu.sync_copy(valv, deg_sh.at[pl.ds(s * ROWS_PER_SUB + k * KC, KC)])

    @pl.loop(0, KC)
    def _ones_val(i):
        @pl.loop(0, D // 16)
        def _ov(j):
            valv[i, pl.ds(j * 16, 16)] = jnp.ones((16,), jnp.float32)

    plsc.subcore_barrier()

    @pl.loop(0, NCHUNK)
    def _accumulate(j):
        pltpu.sync_copy(valv, deg_sh.at[dstv.at[j]], add=True)

    plsc.subcore_barrier()
    pltpu.sync_copy(
        deg_sh.at[pl.ds(s * ROWS_PER_SUB, ROWS_PER_SUB)],
        out_hbm.at[pl.ds(c * N_PAD + s * ROWS_PER_SUB, ROWS_PER_SUB)],
    )


@functools.partial(
    pl.kernel,
    out_type=jax.ShapeDtypeStruct((NC * N_PAD, D), jnp.float32),
    mesh=_MESH,
    scratch_types=(
        [pltpu.VMEM((_GB, 2, KC), jnp.int32) for _ in range(2)]
        + [pltpu.VMEM((KC, D), jnp.float32) for _ in range(2)]
        + [pltpu.SemaphoreType.DMA for _ in range(4)]
        + [pltpu.VMEM_SHARED((N_PAD, D), jnp.float32)]
    ),
)
def _agg_kernel(table_hbm, idx_hbm, out_hbm, ib0, ib1, r0, r1,
                isem0, isem1, gsem0, gsem1, acc_sh):
    idxb = (ib0, ib1)
    isem = (isem0, isem1)
    rows = (r0, r1)
    gsem = (gsem0, gsem1)

    c = lax.axis_index("c")
    s = lax.axis_index("s")

    @pl.loop(0, KC)
    def _zero_rows(i):
        @pl.loop(0, D // 16)
        def _zero_lane(j):
            rows[0][i, pl.ds(j * 16, 16)] = jnp.zeros((16,), jnp.float32)

    @pl.loop(0, ROWS_PER_SUB // KC)
    def _zero_acc(k):
        pltpu.sync_copy(rows[0], acc_sh.at[pl.ds(s * ROWS_PER_SUB + k * KC, KC)])

    plsc.subcore_barrier()

    def _run(cb, nblk):
        def iblk(k):
            return idx_hbm.at[pl.ds(cb + k * _GB, _GB)]

        for p in range(2):
            pltpu.async_copy(iblk(p), idxb[p], isem[p])
        pltpu.make_async_copy(iblk(0), idxb[0], isem[0]).wait()
        for b in range(2):
            pltpu.async_copy(table_hbm.at[idxb[0].at[b, 0]], rows[b], gsem[b])

        def _block(k, p, has_next, prefetch, tail):
            q = 1 - p
            if has_next:
                pltpu.make_async_copy(iblk(k + 1), idxb[q], isem[q]).wait()
            for b in range(_GB):
                rb = rows[b % 2]
                pltpu.make_async_copy(
                    table_hbm.at[idxb[p].at[b, 0]], rb, gsem[b % 2]).wait()
                pltpu.sync_copy(rb, acc_sh.at[idxb[p].at[b, 1]], add=True)
                if not (tail and b >= _GB - 2):
                    if b < _GB - 2:
                        nsrc = idxb[p].at[b + 2, 0]
                    else:
                        nsrc = idxb[q].at[b - (_GB - 2), 0]
                    pltpu.async_copy(table_hbm.at[nsrc], rb, gsem[b % 2])
            if prefetch:
                pltpu.async_copy(iblk(k + 2), idxb[p], isem[p])

        @pl.loop(0, nblk - 2, step=2)
        def _blocks(m):
            _block(m, 0, True, True, False)
            _block(m + 1, 1, True, True, False)

        _block(nblk - 2, 0, True, False, False)
        _block(nblk - 1, 1, False, False, True)

    @pl.when(c == 0)
    def _core0():
        _run(s * _NCH0, _NBLK0)

    @pl.when(c == 1)
    def _core1():
        _run(_C1_BASE + s * _NCH1, _NBLK1)

    plsc.subcore_barrier()
    pltpu.sync_copy(
        acc_sh.at[pl.ds(s * ROWS_PER_SUB, ROWS_PER_SUB)],
        out_hbm.at[pl.ds(c * N_PAD + s * ROWS_PER_SUB, ROWS_PER_SUB)],
    )


_RB = 1000


def _dinv_from_parts(degp):
    deg = degp[0, :, 0:1] + degp[1, :, 0:1]
    return jnp.where(deg > 0, lax.rsqrt(jnp.maximum(deg, 1e-12)), 0.0)


def _mm1_body(x_ref, w_ref, degp_ref, o_ref):
    dinv = _dinv_from_parts(degp_ref[...])
    h = jnp.dot(x_ref[...], w_ref[...], preferred_element_type=jnp.float32)
    o_ref[...] = h * dinv


def _mm1(x, W1, degp):
    return pl.pallas_call(
        _mm1_body,
        grid=(N_NODES // _RB,),
        in_specs=[
            pl.BlockSpec((_RB, D), lambda i: (i, 0)),
            pl.BlockSpec((D, D), lambda i: (0, 0)),
            pl.BlockSpec((NC, _RB, D), lambda i: (0, i, 0)),
        ],
        out_specs=pl.BlockSpec((_RB, D), lambda i: (i, 0)),
        out_shape=jax.ShapeDtypeStruct((N_NODES, D), jnp.float32),
    )(x, W1, degp)


def _mid_body(p_ref, p2_ref, degp_ref, b_ref, w_ref, o_ref):
    dinv = _dinv_from_parts(degp_ref[...])
    agg = p_ref[0] + p_ref[1]
    h = jax.nn.relu(agg * dinv + b_ref[...])
    o_ref[...] = (
        jnp.dot(h, w_ref[...], preferred_element_type=jnp.float32) * dinv
    )


def _mid(P, P2, degp, b1, W2):
    return pl.pallas_call(
        _mid_body,
        grid=(N_NODES // _RB,),
        in_specs=[
            pl.BlockSpec((NC, _RB, D), lambda i: (0, i, 0)),
            pl.BlockSpec((NC, _RB, D), lambda i: (0, i, 0)),
            pl.BlockSpec((NC, _RB, D), lambda i: (0, i, 0)),
            pl.BlockSpec((1, D), lambda i: (0, 0)),
            pl.BlockSpec((D, D), lambda i: (0, 0)),
        ],
        out_specs=pl.BlockSpec((_RB, D), lambda i: (i, 0)),
        out_shape=jax.ShapeDtypeStruct((N_NODES, D), jnp.float32),
    )(P, P2, degp, b1, W2)


def _fin_body(q_ref, q2_ref, degp_ref, b_ref, wih_ref, bih_ref, bhh_ref,
              o_ref):
    dinv = _dinv_from_parts(degp_ref[...])
    agg = q_ref[0] + q_ref[1]
    h = jax.nn.relu(agg * dinv + b_ref[...])
    g = jnp.dot(h, wih_ref[...], preferred_element_type=jnp.float32)
    g = g + bih_ref[...]
    xr = g[:, 0:D]
    xz = g[:, D:2 * D]
    xn = g[:, 2 * D:3 * D]
    hr = bhh_ref[:, 0:D]
    hz = bhh_ref[:, D:2 * D]
    hn = bhh_ref[:, 2 * D:3 * D]
    r = jax.nn.sigmoid(xr + hr)
    z = jax.nn.sigmoid(xz + hz)
    n = jnp.tanh(xn + r * hn)
    o_ref[...] = (1.0 - z) * n


def _fin(Q, Q2, degp, b2, W_ih, b_ih, b_hh):
    return pl.pallas_call(
        _fin_body,
        grid=(N_NODES // _RB,),
        in_specs=[
            pl.BlockSpec((NC, _RB, D), lambda i: (0, i, 0)),
            pl.BlockSpec((NC, _RB, D), lambda i: (0, i, 0)),
            pl.BlockSpec((NC, _RB, D), lambda i: (0, i, 0)),
            pl.BlockSpec((1, D), lambda i: (0, 0)),
            pl.BlockSpec((D, 3 * D), lambda i: (0, 0)),
            pl.BlockSpec((1, 3 * D), lambda i: (0, 0)),
            pl.BlockSpec((1, 3 * D), lambda i: (0, 0)),
        ],
        out_specs=pl.BlockSpec((_RB, D), lambda i: (i, 0)),
        out_shape=jax.ShapeDtypeStruct((N_NODES, D), jnp.float32),
    )(Q, Q2, degp, b2, W_ih, b_ih, b_hh)


def kernel(x, A, W1, b1, W2, b2, W_ih, W_hh, b_ih, b_hh):
    A = A.astype(jnp.int32)
    src = A[0]
    dst = A[1]
    pad = E_PAD - src.shape[0]
    pad_dst = DUMMY + jnp.arange(pad, dtype=jnp.int32) % (N_PAD - N_NODES)
    pad_src = jnp.arange(pad, dtype=jnp.int32) * 13 % N_NODES
    src_p = jnp.concatenate([src, pad_src])
    dst_p = jnp.concatenate([dst, pad_dst])
    idx = jnp.stack([src_p.reshape(CHUNKS, KC),
                     dst_p.reshape(CHUNKS, KC)], axis=1)
    dst_p = dst_p.reshape(CHUNKS, KC)

    degp = _deg_kernel(dst_p).reshape(NC, N_PAD, D)

    hs1 = _mm1(x, W1, degp)
    P = _agg_kernel(hs1, idx).reshape(NC, N_PAD, D)
    hs2 = _mid(P, P, degp, b1.reshape(1, D), W2)
    Q = _agg_kernel(hs2, idx).reshape(NC, N_PAD, D)
    out = _fin(Q, Q, degp, b2.reshape(1, D), W_ih, b_ih.reshape(1, 3 * D),
               b_hh.reshape(1, 3 * D))
    return out
